# baseline (speedup 1.0000x reference)
import functools

import jax
import jax.numpy as jnp
from jax import lax
from jax.experimental import pallas as pl
from jax.experimental.pallas import tpu as pltpu
from jax.experimental.pallas import tpu_sc as plsc

_N = 10000
_NP = 10240
_D = 128
_E = 320000
_L = 100000
_LT = 200000
_LP = 204800

_NC = 2
_NS = 16
_NW = _NC * _NS
_KC = 128
_ECH = 2560
_EP = _KC * _ECH
_RPT = _NP // _NS
_DK = 80
_DCH = _LP // _DK
_NB = 2
_NBD = 2
_K0 = 160
_K1 = 0

_BLK = 1024
_GRID = _NP // _BLK


def _sc_mesh():
    return plsc.VectorSubcoreMesh(core_axis_name="c", subcore_axis_name="s")


def _chunk_range(c, s):
    base = jnp.where(c == 0, s * _K0, 0)
    cnt = jnp.where(c == 0, _K0, _K1)
    return base, cnt


def _stage(c, base, hbm, vmem):
    @pl.when(c == 0)
    def _():
        pltpu.sync_copy(hbm.at[pl.ds(base, _K0)], vmem.at[pl.ds(0, _K0)])


def _deg_body(dst_hbm, w_hbm, degp_hbm, didx, wbuf, zb, acc, sem):
    c = lax.axis_index("c")
    s = lax.axis_index("s")
    base, cnt = _chunk_range(c, s)
    _stage(c, base, dst_hbm, didx)
    _stage(c, base, w_hbm, wbuf)

    def zstore(i, carry):
        zb[pl.ds(i * 16, 16)] = jnp.zeros((16,), jnp.float32)
        return carry

    lax.fori_loop(0, _RPT // 16, zstore, 0)
    pltpu.sync_copy(zb, acc.at[pl.ds(s * _RPT, _RPT)])
    plsc.subcore_barrier()

    def fire(ci, carry):
        pltpu.async_copy(wbuf.at[ci], acc.at[didx.at[ci]], sem, add=True)
        return carry

    lax.fori_loop(0, cnt, fire, 0)

    def drain(ci, carry):
        pltpu.make_async_copy(wbuf.at[0], acc.at[didx.at[0]], sem).wait()
        return carry

    lax.fori_loop(0, cnt, drain, 0)
    plsc.subcore_barrier()
    pltpu.sync_copy(acc.at[pl.ds(s * _RPT, _RPT)],
                    degp_hbm.at[c, pl.ds(s * _RPT, _RPT)])


_deg_call = pl.kernel(
    _deg_body,
    out_type=jax.ShapeDtypeStruct((_NC, _NP), jnp.float32),
    mesh=_sc_mesh(),
    scratch_types=[
        pltpu.VMEM((_K0, _KC), jnp.int32),
        pltpu.VMEM((_K0, _KC), jnp.float32),
        pltpu.VMEM((_RPT,), jnp.float32),
        pltpu.VMEM_SHARED((_NP,), jnp.float32),
        pltpu.SemaphoreType.DMA,
    ],
)


_DH = _D // 2


def _edge_body(yl_hbm, yr_hbm, src_hbm, dst_hbm, w_hbm, ppl_hbm, ppr_hbm,
               *scr):
    sidx, didx = scr[0], scr[1]
    rgs = scr[2:2 + _NB]
    rss = scr[2 + _NB:2 + 2 * _NB]
    wrs = scr[2 + 2 * _NB:2 + 3 * _NB]
    zb, acc = scr[2 + 3 * _NB], scr[3 + 3 * _NB]
    gsems = scr[4 + 3 * _NB:4 + 4 * _NB]
    ssems = scr[4 + 4 * _NB:4 + 5 * _NB]
    wsems = scr[4 + 5 * _NB:4 + 6 * _NB]
    c = lax.axis_index("c")
    s = lax.axis_index("s")
    base, cnt = _chunk_range(c, s)
    with jax.named_scope("edge_stage"):
        _stage(c, base, src_hbm, sidx)
        _stage(c, base, dst_hbm, didx)

    def zstore(r, carry):
        for v in range(_DH // 16):
            zb[r, pl.ds(v * 16, 16)] = jnp.zeros((16,), jnp.float32)
        return carry

    lax.fori_loop(0, _KC, zstore, 0)

    for half, (y_hbm, pp_hbm) in enumerate(((yl_hbm, ppl_hbm),
                                            (yr_hbm, ppr_hbm))):
        if half:
            plsc.subcore_barrier()
        with jax.named_scope("edge_zero"):
            for j in range(_RPT // _KC):
                pltpu.sync_copy(zb, acc.at[pl.ds(s * _RPT + j * _KC, _KC)])
            plsc.subcore_barrier()

        @pl.when(c == 0)
        def _():
            for b in range(_NB):
                pltpu.async_copy(y_hbm.at[sidx.at[b]], rgs[b], gsems[b])
                pltpu.async_copy(w_hbm.at[base + b], wrs[b], wsems[b])

        def step(i, carry):
            for b in range(_NB):
                ci = i * _NB + b
                pltpu.make_async_copy(y_hbm.at[sidx.at[ci]],
                                      rgs[b], gsems[b]).wait()
                pltpu.make_async_copy(w_hbm.at[base], wrs[b], wsems[b]).wait()

                @pl.when(i > 0)
                def _():
                    pltpu.make_async_copy(rss[b], acc.at[didx.at[ci]],
                                          ssems[b]).wait()

                def scale(g, inner):
                    wv = wrs[b][pl.ds(g * 16, 16)]
                    for e in range(16):
                        wb = wv[e]
                        r = g * 16 + e
                        for v in range(_DH // 16):
                            sl = pl.ds(v * 16, 16)
                            rss[b][r, sl] = rgs[b][r, sl] * wb
                    return inner

                lax.fori_loop(0, _KC // 16, scale, 0)

                cn = ci + _NB

                @pl.when(cn < cnt)
                def _():
                    pltpu.async_copy(y_hbm.at[sidx.at[cn]], rgs[b], gsems[b])
                    pltpu.async_copy(w_hbm.at[base + cn], wrs[b], wsems[b])

                pltpu.async_copy(rss[b], acc.at[didx.at[ci]], ssems[b],
                                 add=True)
            return carry

        with jax.named_scope("edge_main"):
            lax.fori_loop(0, cnt // _NB, step, 0)

            @pl.when(c == 0)
            def _():
                for b in range(_NB):
                    pltpu.make_async_copy(rss[b], acc.at[didx.at[0]],
                                          ssems[b]).wait()

            plsc.subcore_barrier()
        with jax.named_scope("edge_dump"):
            pltpu.sync_copy(acc.at[pl.ds(s * _RPT, _RPT)],
                            pp_hbm.at[c, pl.ds(s * _RPT, _RPT)])


_edge_call = pl.kernel(
    _edge_body,
    out_type=(jax.ShapeDtypeStruct((_NC, _NP, _DH), jnp.float32),
              jax.ShapeDtypeStruct((_NC, _NP, _DH), jnp.float32)),
    mesh=_sc_mesh(),
    scratch_types=(
        [pltpu.VMEM((_K0, _KC), jnp.int32)] * 2
        + [pltpu.VMEM((_KC, _DH), jnp.float32)] * (2 * _NB)
        + [pltpu.VMEM((_KC,), jnp.float32)] * _NB
        + [pltpu.VMEM((_KC, _DH), jnp.float32)]
        + [pltpu.VMEM_SHARED((_NP, _DH), jnp.float32)]
        + [pltpu.SemaphoreType.DMA] * (3 * _NB)
    ),
    compiler_params=pltpu.CompilerParams(use_tc_tiling_on_sc=False),
)


def _dec_body(enc_hbm, ia_hbm, ib_hbm, out_hbm,
              iav, ibv, ar0, ar1, br0, br1, obuf,
              sema0, sema1, semb0, semb1):
    c = lax.axis_index("c")
    s = lax.axis_index("s")
    base, cnt = _chunk_range(c, s)
    _stage(c, base, ia_hbm, iav)
    _stage(c, base, ib_hbm, ibv)
    lanes = jnp.arange(16, dtype=jnp.int32)
    ars = (ar0, ar1)
    brs = (br0, br1)
    semas = (sema0, sema1)
    sembs = (semb0, semb1)

    @pl.when(c == 0)
    def _():
        for b in range(_NBD):
            pltpu.async_copy(enc_hbm.at[iav.at[b]], ars[b], semas[b])
            pltpu.async_copy(enc_hbm.at[ibv.at[b]], brs[b], sembs[b])

    def chunk(i, carry):
        for b in range(_NBD):
            ci = i * _NBD + b
            pltpu.make_async_copy(enc_hbm.at[iav.at[ci]],
                                  ars[b], semas[b]).wait()
            pltpu.make_async_copy(enc_hbm.at[ibv.at[ci]],
                                  brs[b], sembs[b]).wait()
            for g in range(_DK // 16):
                rid = g * 16 + lanes

                def dstep(dj, acc):
                    for u in range(8):
                        cidx = (lanes + (dj * 8 + u)) & 127
                        av = plsc.load_gather(ars[b], [rid, cidx])
                        bv = plsc.load_gather(brs[b], [rid, cidx])
                        acc = acc + av * bv
                    return acc

                acc = lax.fori_loop(0, _D // 8, dstep,
                                    jnp.zeros((16,), jnp.float32))
                obuf[ci, pl.ds(g * 16, 16)] = acc
            cn = ci + _NBD

            @pl.when(cn < cnt)
            def _():
                pltpu.async_copy(enc_hbm.at[iav.at[cn]], ars[b], semas[b])
                pltpu.async_copy(enc_hbm.at[ibv.at[cn]], brs[b], sembs[b])

        return carry

    lax.fori_loop(0, cnt // _NBD, chunk, 0)

    @pl.when(c == 0)
    def _():
        pltpu.sync_copy(obuf.at[pl.ds(0, _K0)], out_hbm.at[pl.ds(base, _K0)])


_dec_call = pl.kernel(
    _dec_body,
    out_type=jax.ShapeDtypeStruct((_DCH, _DK), jnp.float32),
    mesh=_sc_mesh(),
    scratch_types=[
        pltpu.VMEM((_K0, _DK), jnp.int32),
        pltpu.VMEM((_K0, _DK), jnp.int32),
        pltpu.VMEM((_DK, _D), jnp.float32),
        pltpu.VMEM((_DK, _D), jnp.float32),
        pltpu.VMEM((_DK, _D), jnp.float32),
        pltpu.VMEM((_DK, _D), jnp.float32),
        pltpu.VMEM((_K0, _DK), jnp.float32),
        pltpu.SemaphoreType.DMA,
        pltpu.SemaphoreType.DMA,
        pltpu.SemaphoreType.DMA,
        pltpu.SemaphoreType.DMA,
    ],
    compiler_params=pltpu.CompilerParams(needs_layout_passes=False),
)


def _dinv_of(deg_col):
    deg = deg_col[:, 0:1] + deg_col[:, 1:2] + 1.0
    return jnp.where(deg > 0, lax.rsqrt(jnp.maximum(deg, 1e-12)), 0.0)


def _tc_first_body(degt_ref, x_ref, w_ref, ol_ref, or_ref):
    dinv = _dinv_of(degt_ref[...])
    y = dinv * jnp.dot(x_ref[...], w_ref[...],
                       preferred_element_type=jnp.float32)
    ol_ref[...] = y[:, :_DH]
    or_ref[...] = y[:, _DH:]


def _tc_mid_body(degt_ref, ppl_ref, ppr_ref, yl_ref, yr_ref, b_ref, w_ref,
                 ol_ref, or_ref):
    dinv = _dinv_of(degt_ref[...])
    pl_ = ppl_ref[...]
    pr_ = ppr_ref[...]
    hl = pl_[0] + pl_[1] + yl_ref[...]
    hr = pr_[0] + pr_[1] + yr_ref[...]
    h = dinv * jnp.concatenate([hl, hr], axis=-1) + b_ref[...]
    h = jnp.maximum(h, 0.0)
    y = dinv * jnp.dot(h, w_ref[...], preferred_element_type=jnp.float32)
    ol_ref[...] = y[:, :_DH]
    or_ref[...] = y[:, _DH:]


def _tc_enc_body(degt_ref, ppl_ref, ppr_ref, yl_ref, yr_ref, b_ref, o_ref):
    dinv = _dinv_of(degt_ref[...])
    pl_ = ppl_ref[...]
    pr_ = ppr_ref[...]
    hl = pl_[0] + pl_[1] + yl_ref[...]
    hr = pr_[0] + pr_[1] + yr_ref[...]
    o_ref[...] = dinv * jnp.concatenate([hl, hr], axis=-1) + b_ref[...]


_deg_spec = pl.BlockSpec((_BLK, 2), lambda i: (i, 0))
_row_spec = pl.BlockSpec((_BLK, _D), lambda i: (i, 0))
_half_spec = pl.BlockSpec((_BLK, _DH), lambda i: (i, 0))
_pp_spec = pl.BlockSpec((2, _BLK, _DH), lambda i: (0, i, 0))
_w_spec = pl.BlockSpec((_D, _D), lambda i: (0, 0))
_b_spec = pl.BlockSpec((1, _D), lambda i: (0, 0))
_out_struct = jax.ShapeDtypeStruct((_NP, _D), jnp.float32)
_half_struct = jax.ShapeDtypeStruct((_NP, _DH), jnp.float32)

_tc_first = pl.pallas_call(
    _tc_first_body,
    grid=(_GRID,),
    in_specs=[_deg_spec, _row_spec, _w_spec],
    out_specs=(_half_spec, _half_spec),
    out_shape=(_half_struct, _half_struct),
)

_tc_mid = pl.pallas_call(
    _tc_mid_body,
    grid=(_GRID,),
    in_specs=[_deg_spec, _pp_spec, _pp_spec, _half_spec, _half_spec,
              _b_spec, _w_spec],
    out_specs=(_half_spec, _half_spec),
    out_shape=(_half_struct, _half_struct),
)

_tc_enc = pl.pallas_call(
    _tc_enc_body,
    grid=(_GRID,),
    in_specs=[_deg_spec, _pp_spec, _pp_spec, _half_spec, _half_spec, _b_spec],
    out_specs=_row_spec,
    out_shape=_out_struct,
)


def kernel(x, edge_index, edge_weight, edge_label_index,
           W1, b1, W2, b2, W3, b3, W4, b4, W5, b5):
    src3 = jnp.pad(edge_index[0], (0, _EP - _E),
                   constant_values=_N).reshape(_ECH, _KC)
    dst3 = jnp.pad(edge_index[1], (0, _EP - _E),
                   constant_values=_N).reshape(_ECH, _KC)
    w3 = jnp.pad(edge_weight, (0, _EP - _E)).reshape(_ECH, _KC)
    xp = jnp.pad(x, ((0, _NP - _N), (0, 0)))

    neg = jax.random.randint(jax.random.key(12345), (2, _L), 0, _N,
                             dtype=edge_index.dtype)
    eli = jnp.concatenate([edge_label_index, neg], axis=1)
    eli = jnp.pad(eli, ((0, 0), (0, _LP - _LT)))
    ia = eli[0].reshape(_DCH, _DK)
    ib = eli[1].reshape(_DCH, _DK)

    degp = _deg_call(dst3, w3)
    degt = degp.T

    ws = [W1, W2, W3, W4, W5]
    bs = [b.reshape(1, _D) for b in (b1, b2, b3, b4, b5)]

    yl, yr = _tc_first(degt, xp, ws[0])
    for l in range(4):
        ppl, ppr = _edge_call(yl, yr, src3, dst3, w3)
        yl, yr = _tc_mid(degt, ppl, ppr, yl, yr, bs[l], ws[l + 1])
    ppl, ppr = _edge_call(yl, yr, src3, dst3, w3)
    enc = _tc_enc(degt, ppl, ppr, yl, yr, bs[4])

    dec = _dec_call(enc, ia, ib)
    return dec.reshape(_LP)[:_LT]


# speedup vs baseline: 1.4613x; 1.4613x over previous
import functools

import jax
import jax.numpy as jnp
from jax import lax
from jax.experimental import pallas as pl
from jax.experimental.pallas import tpu as pltpu
from jax.experimental.pallas import tpu_sc as plsc

_N = 10000
_NP = 10240
_D = 128
_E = 320000
_L = 100000
_LT = 200000
_LP = 204800

_NC = 2
_NS = 16
_NW = _NC * _NS
_KC = 128
_ECH = 2560
_EP = _KC * _ECH
_RPT = _NP // _NS
_DK = 80
_DCH = _LP // _DK
_NB = 2
_NBD = 2
_K0 = 144
_K1 = 16
_C1BASE = _NS * _K0

_BLK = 1024
_GRID = _NP // _BLK


def _sc_mesh():
    return plsc.VectorSubcoreMesh(core_axis_name="c", subcore_axis_name="s")


def _chunk_range(c, s):
    base = jnp.where(c == 0, s * _K0, _C1BASE + s * _K1)
    cnt = jnp.where(c == 0, _K0, _K1)
    return base, cnt


def _stage(c, base, hbm, vmem):
    @pl.when(c == 0)
    def _():
        pltpu.sync_copy(hbm.at[pl.ds(base, _K0)], vmem.at[pl.ds(0, _K0)])

    @pl.when(c != 0)
    def _():
        pltpu.sync_copy(hbm.at[pl.ds(base, _K1)], vmem.at[pl.ds(0, _K1)])


def _deg_body(dst_hbm, w_hbm, degp_hbm, didx, wbuf, zb, acc, sem):
    c = lax.axis_index("c")
    s = lax.axis_index("s")
    base, cnt = _chunk_range(c, s)
    _stage(c, base, dst_hbm, didx)
    _stage(c, base, w_hbm, wbuf)

    def zstore(i, carry):
        zb[pl.ds(i * 16, 16)] = jnp.zeros((16,), jnp.float32)
        return carry

    lax.fori_loop(0, _RPT // 16, zstore, 0)
    pltpu.sync_copy(zb, acc.at[pl.ds(s * _RPT, _RPT)])
    plsc.subcore_barrier()

    def fire(ci, carry):
        pltpu.async_copy(wbuf.at[ci], acc.at[didx.at[ci]], sem, add=True)
        return carry

    lax.fori_loop(0, cnt, fire, 0)

    def drain(ci, carry):
        pltpu.make_async_copy(wbuf.at[0], acc.at[didx.at[0]], sem).wait()
        return carry

    lax.fori_loop(0, cnt, drain, 0)
    plsc.subcore_barrier()
    pltpu.sync_copy(acc.at[pl.ds(s * _RPT, _RPT)],
                    degp_hbm.at[c, pl.ds(s * _RPT, _RPT)])


_deg_call = pl.kernel(
    _deg_body,
    out_type=jax.ShapeDtypeStruct((_NC, _NP), jnp.float32),
    mesh=_sc_mesh(),
    scratch_types=[
        pltpu.VMEM((_K0, _KC), jnp.int32),
        pltpu.VMEM((_K0, _KC), jnp.float32),
        pltpu.VMEM((_RPT,), jnp.float32),
        pltpu.VMEM_SHARED((_NP,), jnp.float32),
        pltpu.SemaphoreType.DMA,
    ],
)


_DH = _D // 2


def _edge_body(yl_hbm, yr_hbm, src_hbm, dst_hbm, w_hbm, ppl_hbm, ppr_hbm,
               *scr):
    sidx, didx = scr[0], scr[1]
    rgs = scr[2:2 + _NB]
    rss = scr[2 + _NB:2 + 2 * _NB]
    wrs = scr[2 + 2 * _NB:2 + 3 * _NB]
    zb, acc = scr[2 + 3 * _NB], scr[3 + 3 * _NB]
    gsems = scr[4 + 3 * _NB:4 + 4 * _NB]
    ssems = scr[4 + 4 * _NB:4 + 5 * _NB]
    wsems = scr[4 + 5 * _NB:4 + 6 * _NB]
    c = lax.axis_index("c")
    s = lax.axis_index("s")
    base, cnt = _chunk_range(c, s)
    with jax.named_scope("edge_stage"):
        _stage(c, base, src_hbm, sidx)
        _stage(c, base, dst_hbm, didx)

    def zstore(r, carry):
        for v in range(_DH // 16):
            zb[r, pl.ds(v * 16, 16)] = jnp.zeros((16,), jnp.float32)
        return carry

    lax.fori_loop(0, _KC, zstore, 0)

    for half, (y_hbm, pp_hbm) in enumerate(((yl_hbm, ppl_hbm),
                                            (yr_hbm, ppr_hbm))):
        if half:
            plsc.subcore_barrier()
        with jax.named_scope("edge_zero"):
            for j in range(_RPT // _KC):
                pltpu.sync_copy(zb, acc.at[pl.ds(s * _RPT + j * _KC, _KC)])
            plsc.subcore_barrier()

        for b in range(_NB):
            pltpu.async_copy(y_hbm.at[sidx.at[b]], rgs[b], gsems[b])
            pltpu.async_copy(w_hbm.at[base + b], wrs[b], wsems[b])

        def step(i, carry):
            for b in range(_NB):
                ci = i * _NB + b
                pltpu.make_async_copy(y_hbm.at[sidx.at[ci]],
                                      rgs[b], gsems[b]).wait()
                pltpu.make_async_copy(w_hbm.at[base], wrs[b], wsems[b]).wait()

                @pl.when(i > 0)
                def _():
                    pltpu.make_async_copy(rss[b], acc.at[didx.at[ci]],
                                          ssems[b]).wait()

                def scale(g, inner):
                    wv = wrs[b][pl.ds(g * 16, 16)]
                    for e in range(16):
                        wb = wv[e]
                        r = g * 16 + e
                        for v in range(_DH // 16):
                            sl = pl.ds(v * 16, 16)
                            rss[b][r, sl] = rgs[b][r, sl] * wb
                    return inner

                lax.fori_loop(0, _KC // 16, scale, 0)

                cn = ci + _NB

                @pl.when(cn < cnt)
                def _():
                    pltpu.async_copy(y_hbm.at[sidx.at[cn]], rgs[b], gsems[b])
                    pltpu.async_copy(w_hbm.at[base + cn], wrs[b], wsems[b])

                pltpu.async_copy(rss[b], acc.at[didx.at[ci]], ssems[b],
                                 add=True)
            return carry

        with jax.named_scope("edge_main"):
            lax.fori_loop(0, cnt // _NB, step, 0)
            for b in range(_NB):
                pltpu.make_async_copy(rss[b], acc.at[didx.at[0]],
                                      ssems[b]).wait()
            plsc.subcore_barrier()
        with jax.named_scope("edge_dump"):
            pltpu.sync_copy(acc.at[pl.ds(s * _RPT, _RPT)],
                            pp_hbm.at[c, pl.ds(s * _RPT, _RPT)])


_edge_call = pl.kernel(
    _edge_body,
    out_type=(jax.ShapeDtypeStruct((_NC, _NP, _DH), jnp.float32),
              jax.ShapeDtypeStruct((_NC, _NP, _DH), jnp.float32)),
    mesh=_sc_mesh(),
    scratch_types=(
        [pltpu.VMEM((_K0, _KC), jnp.int32)] * 2
        + [pltpu.VMEM((_KC, _DH), jnp.float32)] * (2 * _NB)
        + [pltpu.VMEM((_KC,), jnp.float32)] * _NB
        + [pltpu.VMEM((_KC, _DH), jnp.float32)]
        + [pltpu.VMEM_SHARED((_NP, _DH), jnp.float32)]
        + [pltpu.SemaphoreType.DMA] * (3 * _NB)
    ),
    compiler_params=pltpu.CompilerParams(use_tc_tiling_on_sc=False),
)


def _dec_body(enc_hbm, ia_hbm, ib_hbm, out_hbm,
              iav, ibv, ar0, ar1, br0, br1, obuf,
              sema0, sema1, semb0, semb1):
    c = lax.axis_index("c")
    s = lax.axis_index("s")
    base, cnt = _chunk_range(c, s)
    _stage(c, base, ia_hbm, iav)
    _stage(c, base, ib_hbm, ibv)
    lanes = jnp.arange(16, dtype=jnp.int32)
    ars = (ar0, ar1)
    brs = (br0, br1)
    semas = (sema0, sema1)
    sembs = (semb0, semb1)

    for b in range(_NBD):
        pltpu.async_copy(enc_hbm.at[iav.at[b]], ars[b], semas[b])
        pltpu.async_copy(enc_hbm.at[ibv.at[b]], brs[b], sembs[b])

    def chunk(i, carry):
        for b in range(_NBD):
            ci = i * _NBD + b
            pltpu.make_async_copy(enc_hbm.at[iav.at[ci]],
                                  ars[b], semas[b]).wait()
            pltpu.make_async_copy(enc_hbm.at[ibv.at[ci]],
                                  brs[b], sembs[b]).wait()
            for g in range(_DK // 16):
                rid = g * 16 + lanes

                def dstep(dj, acc):
                    for u in range(8):
                        cidx = (lanes + (dj * 8 + u)) & 127
                        av = plsc.load_gather(ars[b], [rid, cidx])
                        bv = plsc.load_gather(brs[b], [rid, cidx])
                        acc = acc + av * bv
                    return acc

                acc = lax.fori_loop(0, _D // 8, dstep,
                                    jnp.zeros((16,), jnp.float32))
                obuf[ci, pl.ds(g * 16, 16)] = acc
            cn = ci + _NBD

            @pl.when(cn < cnt)
            def _():
                pltpu.async_copy(enc_hbm.at[iav.at[cn]], ars[b], semas[b])
                pltpu.async_copy(enc_hbm.at[ibv.at[cn]], brs[b], sembs[b])

        return carry

    lax.fori_loop(0, cnt // _NBD, chunk, 0)

    @pl.when(c == 0)
    def _():
        pltpu.sync_copy(obuf.at[pl.ds(0, _K0)], out_hbm.at[pl.ds(base, _K0)])

    @pl.when(c != 0)
    def _():
        pltpu.sync_copy(obuf.at[pl.ds(0, _K1)], out_hbm.at[pl.ds(base, _K1)])


_dec_call = pl.kernel(
    _dec_body,
    out_type=jax.ShapeDtypeStruct((_DCH, _DK), jnp.float32),
    mesh=_sc_mesh(),
    scratch_types=[
        pltpu.VMEM((_K0, _DK), jnp.int32),
        pltpu.VMEM((_K0, _DK), jnp.int32),
        pltpu.VMEM((_DK, _D), jnp.float32),
        pltpu.VMEM((_DK, _D), jnp.float32),
        pltpu.VMEM((_DK, _D), jnp.float32),
        pltpu.VMEM((_DK, _D), jnp.float32),
        pltpu.VMEM((_K0, _DK), jnp.float32),
        pltpu.SemaphoreType.DMA,
        pltpu.SemaphoreType.DMA,
        pltpu.SemaphoreType.DMA,
        pltpu.SemaphoreType.DMA,
    ],
    compiler_params=pltpu.CompilerParams(needs_layout_passes=False),
)


def _dinv_of(deg_col):
    deg = deg_col[:, 0:1] + deg_col[:, 1:2] + 1.0
    return jnp.where(deg > 0, lax.rsqrt(jnp.maximum(deg, 1e-12)), 0.0)


def _tc_first_body(degt_ref, x_ref, w_ref, ol_ref, or_ref):
    dinv = _dinv_of(degt_ref[...])
    y = dinv * jnp.dot(x_ref[...], w_ref[...],
                       preferred_element_type=jnp.float32)
    ol_ref[...] = y[:, :_DH]
    or_ref[...] = y[:, _DH:]


def _tc_mid_body(degt_ref, ppl_ref, ppr_ref, yl_ref, yr_ref, b_ref, w_ref,
                 ol_ref, or_ref):
    dinv = _dinv_of(degt_ref[...])
    pl_ = ppl_ref[...]
    pr_ = ppr_ref[...]
    hl = pl_[0] + pl_[1] + yl_ref[...]
    hr = pr_[0] + pr_[1] + yr_ref[...]
    h = dinv * jnp.concatenate([hl, hr], axis=-1) + b_ref[...]
    h = jnp.maximum(h, 0.0)
    y = dinv * jnp.dot(h, w_ref[...], preferred_element_type=jnp.float32)
    ol_ref[...] = y[:, :_DH]
    or_ref[...] = y[:, _DH:]


def _tc_enc_body(degt_ref, ppl_ref, ppr_ref, yl_ref, yr_ref, b_ref, o_ref):
    dinv = _dinv_of(degt_ref[...])
    pl_ = ppl_ref[...]
    pr_ = ppr_ref[...]
    hl = pl_[0] + pl_[1] + yl_ref[...]
    hr = pr_[0] + pr_[1] + yr_ref[...]
    o_ref[...] = dinv * jnp.concatenate([hl, hr], axis=-1) + b_ref[...]


_deg_spec = pl.BlockSpec((_BLK, 2), lambda i: (i, 0))
_row_spec = pl.BlockSpec((_BLK, _D), lambda i: (i, 0))
_half_spec = pl.BlockSpec((_BLK, _DH), lambda i: (i, 0))
_pp_spec = pl.BlockSpec((2, _BLK, _DH), lambda i: (0, i, 0))
_w_spec = pl.BlockSpec((_D, _D), lambda i: (0, 0))
_b_spec = pl.BlockSpec((1, _D), lambda i: (0, 0))
_out_struct = jax.ShapeDtypeStruct((_NP, _D), jnp.float32)
_half_struct = jax.ShapeDtypeStruct((_NP, _DH), jnp.float32)

_tc_first = pl.pallas_call(
    _tc_first_body,
    grid=(_GRID,),
    in_specs=[_deg_spec, _row_spec, _w_spec],
    out_specs=(_half_spec, _half_spec),
    out_shape=(_half_struct, _half_struct),
)

_tc_mid = pl.pallas_call(
    _tc_mid_body,
    grid=(_GRID,),
    in_specs=[_deg_spec, _pp_spec, _pp_spec, _half_spec, _half_spec,
              _b_spec, _w_spec],
    out_specs=(_half_spec, _half_spec),
    out_shape=(_half_struct, _half_struct),
)

_tc_enc = pl.pallas_call(
    _tc_enc_body,
    grid=(_GRID,),
    in_specs=[_deg_spec, _pp_spec, _pp_spec, _half_spec, _half_spec, _b_spec],
    out_specs=_row_spec,
    out_shape=_out_struct,
)


def kernel(x, edge_index, edge_weight, edge_label_index,
           W1, b1, W2, b2, W3, b3, W4, b4, W5, b5):
    src3 = jnp.pad(edge_index[0], (0, _EP - _E),
                   constant_values=_N).reshape(_ECH, _KC)
    dst3 = jnp.pad(edge_index[1], (0, _EP - _E),
                   constant_values=_N).reshape(_ECH, _KC)
    w3 = jnp.pad(edge_weight, (0, _EP - _E)).reshape(_ECH, _KC)
    xp = jnp.pad(x, ((0, _NP - _N), (0, 0)))

    neg = jax.random.randint(jax.random.key(12345), (2, _L), 0, _N,
                             dtype=edge_index.dtype)
    eli = jnp.concatenate([edge_label_index, neg], axis=1)
    eli = jnp.pad(eli, ((0, 0), (0, _LP - _LT)))
    ia = eli[0].reshape(_DCH, _DK)
    ib = eli[1].reshape(_DCH, _DK)

    degp = _deg_call(dst3, w3)
    degt = degp.T

    ws = [W1, W2, W3, W4, W5]
    bs = [b.reshape(1, _D) for b in (b1, b2, b3, b4, b5)]

    yl, yr = _tc_first(degt, xp, ws[0])
    for l in range(4):
        ppl, ppr = _edge_call(yl, yr, src3, dst3, w3)
        yl, yr = _tc_mid(degt, ppl, ppr, yl, yr, bs[l], ws[l + 1])
    ppl, ppr = _edge_call(yl, yr, src3, dst3, w3)
    enc = _tc_enc(degt, ppl, ppr, yl, yr, bs[4])

    dec = _dec_call(enc, ia, ib)
    return dec.reshape(_LP)[:_LT]
